# baseline (speedup 1.0000x reference)
import jax
import jax.numpy as jnp
from jax.experimental import pallas as pl
from jax.experimental.pallas import tpu as pltpu


def _gcn_kernel(adj_ref, f_ref, n_ref, w_ref, b2_ref, alpha_ref, h_ref, s_ref):
    F = w_ref.shape[1]

    @pl.when(pl.program_id(0) == 0)
    def _():
        w = w_ref[:]
        s_ref[:, :F] = jnp.dot(
            f_ref[:], w, preferred_element_type=jnp.float32
        ).astype(jnp.bfloat16)
        s_ref[:, F:] = jnp.dot(
            n_ref[:], w, preferred_element_type=jnp.float32
        ).astype(jnp.bfloat16)

    acc = jnp.dot(adj_ref[:].astype(jnp.bfloat16), s_ref[:],
                  preferred_element_type=jnp.float32)
    h = acc + b2_ref[:]
    alpha = alpha_ref[0, 0]
    h_ref[:] = jnp.where(h >= 0, h, alpha * h).astype(jnp.bfloat16)


def _disc_kernel(h_ref, dwt_ref, db_ref, sc1_ref, sc2_ref):
    n = h_ref.shape[0]
    F = dwt_ref.shape[0]
    hp = h_ref[:, :F].astype(jnp.float32)
    hn = h_ref[:, F:].astype(jnp.float32)
    c = jnp.sum(hp, axis=0, keepdims=True) * (1.0 / n)
    s = jax.nn.sigmoid(c)
    u = jnp.dot(s, dwt_ref[:], preferred_element_type=jnp.float32)
    db = db_ref[0, 0]
    sc1_ref[:] = jnp.sum(hp * u, axis=1, keepdims=True) + db
    sc2_ref[:] = jnp.sum(hn * u, axis=1, keepdims=True) + db


def kernel(features, negative_features, adj, W_gcn, b_gcn, prelu_alpha, disc_W, disc_b):
    B, N, IN_F = features.shape
    OUT_F = W_gcn.shape[1]
    f2 = features.reshape(N, IN_F)
    n2 = negative_features.reshape(N, IN_F)
    adj2 = adj.reshape(N, N)
    b2 = jnp.concatenate([b_gcn, b_gcn]).reshape(1, 2 * OUT_F)
    alpha = prelu_alpha.reshape(1, 1)
    db = disc_b.reshape(1, 1)
    dwt = disc_W.T

    BM = 200
    h = pl.pallas_call(
        _gcn_kernel,
        grid=(N // BM,),
        in_specs=[
            pl.BlockSpec((BM, N), lambda i: (i, 0)),
            pl.BlockSpec((N, IN_F), lambda i: (0, 0)),
            pl.BlockSpec((N, IN_F), lambda i: (0, 0)),
            pl.BlockSpec((IN_F, OUT_F), lambda i: (0, 0)),
            pl.BlockSpec((1, 2 * OUT_F), lambda i: (0, 0)),
            pl.BlockSpec((1, 1), lambda i: (0, 0)),
        ],
        out_specs=pl.BlockSpec((BM, 2 * OUT_F), lambda i: (i, 0)),
        out_shape=jax.ShapeDtypeStruct((N, 2 * OUT_F), jnp.bfloat16),
        scratch_shapes=[pltpu.VMEM((N, 2 * OUT_F), jnp.bfloat16)],
    )(adj2, f2, n2, W_gcn, b2, alpha)

    sc1, sc2 = pl.pallas_call(
        _disc_kernel,
        out_shape=[
            jax.ShapeDtypeStruct((N, 1), jnp.float32),
            jax.ShapeDtypeStruct((N, 1), jnp.float32),
        ],
    )(h, dwt, db)

    return jnp.concatenate([sc1.reshape(1, N), sc2.reshape(1, N)], axis=1)


# speedup vs baseline: 1.0348x; 1.0348x over previous
import jax
import jax.numpy as jnp
from jax.experimental import pallas as pl
from jax.experimental.pallas import tpu as pltpu


def _gcn_kernel(adj_t_ref, adj_b_ref, f_ref, n_ref, w_ref, b2_ref, alpha_ref,
                ht_ref, hb_ref, s_ref):
    F = w_ref.shape[1]

    @pl.when(pl.program_id(0) == 0)
    def _():
        w = w_ref[:]
        s_ref[:, :F] = jnp.dot(
            f_ref[:], w, preferred_element_type=jnp.float32
        ).astype(jnp.bfloat16)
        s_ref[:, F:] = jnp.dot(
            n_ref[:], w, preferred_element_type=jnp.float32
        ).astype(jnp.bfloat16)

    s = s_ref[:]
    b2 = b2_ref[:]
    alpha = alpha_ref[0, 0]

    def mm(a_ref, o_ref):
        acc = jnp.dot(a_ref[0].astype(jnp.bfloat16), s,
                      preferred_element_type=jnp.float32)
        h = acc + b2
        o_ref[0] = jnp.where(h >= 0, h, alpha * h).astype(jnp.bfloat16)

    mm(adj_t_ref, ht_ref)
    mm(adj_b_ref, hb_ref)


def _disc_kernel(ht_ref, hb_ref, dwt_ref, db_ref,
                 sc1t_ref, sc1b_ref, sc2t_ref, sc2b_ref):
    n = ht_ref.shape[0] + hb_ref.shape[0]
    F = dwt_ref.shape[0]
    hpt = ht_ref[:, :F].astype(jnp.float32)
    hnt = ht_ref[:, F:].astype(jnp.float32)
    hpb = hb_ref[:, :F].astype(jnp.float32)
    hnb = hb_ref[:, F:].astype(jnp.float32)
    c = (jnp.sum(hpt, axis=0, keepdims=True)
         + jnp.sum(hpb, axis=0, keepdims=True)) * (1.0 / n)
    sg = jax.nn.sigmoid(c)
    u = jnp.dot(sg, dwt_ref[:], preferred_element_type=jnp.float32)
    db = db_ref[0, 0]
    sc1t_ref[:] = jnp.sum(hpt * u, axis=1, keepdims=True) + db
    sc1b_ref[:] = jnp.sum(hpb * u, axis=1, keepdims=True) + db
    sc2t_ref[:] = jnp.sum(hnt * u, axis=1, keepdims=True) + db
    sc2b_ref[:] = jnp.sum(hnb * u, axis=1, keepdims=True) + db


def kernel(features, negative_features, adj, W_gcn, b_gcn, prelu_alpha, disc_W, disc_b):
    B, N, IN_F = features.shape
    OUT_F = W_gcn.shape[1]
    H = N // 2
    f2 = features.reshape(N, IN_F)
    n2 = negative_features.reshape(N, IN_F)
    adj3 = adj.reshape(2, H, N)
    b2 = jnp.concatenate([b_gcn, b_gcn]).reshape(1, 2 * OUT_F)
    alpha = prelu_alpha.reshape(1, 1)
    db = disc_b.reshape(1, 1)
    dwt = disc_W.T

    BM = 200
    ht, hb = pl.pallas_call(
        _gcn_kernel,
        grid=(H // BM,),
        in_specs=[
            pl.BlockSpec((1, BM, N), lambda i: (0, i, 0)),
            pl.BlockSpec((1, BM, N), lambda i: (1, i, 0)),
            pl.BlockSpec((N, IN_F), lambda i: (0, 0)),
            pl.BlockSpec((N, IN_F), lambda i: (0, 0)),
            pl.BlockSpec((IN_F, OUT_F), lambda i: (0, 0)),
            pl.BlockSpec((1, 2 * OUT_F), lambda i: (0, 0)),
            pl.BlockSpec((1, 1), lambda i: (0, 0)),
        ],
        out_specs=[
            pl.BlockSpec((1, BM, 2 * OUT_F), lambda i: (0, i, 0)),
            pl.BlockSpec((1, BM, 2 * OUT_F), lambda i: (0, i, 0)),
        ],
        out_shape=[
            jax.ShapeDtypeStruct((1, H, 2 * OUT_F), jnp.bfloat16),
            jax.ShapeDtypeStruct((1, H, 2 * OUT_F), jnp.bfloat16),
        ],
        scratch_shapes=[pltpu.VMEM((N, 2 * OUT_F), jnp.bfloat16)],
    )(adj3, adj3, f2, n2, W_gcn, b2, alpha)

    sc1t, sc1b, sc2t, sc2b = pl.pallas_call(
        _disc_kernel,
        out_shape=[
            jax.ShapeDtypeStruct((H, 1), jnp.float32),
            jax.ShapeDtypeStruct((H, 1), jnp.float32),
            jax.ShapeDtypeStruct((H, 1), jnp.float32),
            jax.ShapeDtypeStruct((H, 1), jnp.float32),
        ],
    )(ht.reshape(H, 2 * OUT_F), hb.reshape(H, 2 * OUT_F), dwt, db)

    return jnp.concatenate(
        [sc1t.reshape(1, H), sc1b.reshape(1, H),
         sc2t.reshape(1, H), sc2b.reshape(1, H)], axis=1)
